# baseline (speedup 1.0000x reference)
import functools

import jax
import jax.numpy as jnp
from jax import lax
from jax.experimental import pallas as pl
from jax.experimental.pallas import tpu as pltpu
from jax.experimental.pallas import tpu_sc as plsc

_IMAGE_TOKEN_ID = 1

_N = 16384
_D = 2048
_NC = 2
_NS = 16
_NW = _NC * _NS
_CHUNK = _N // _NW
_L = 16
_GS = 2
_G = 1 << _GS
_NG = _CHUNK // _G
_K = 12
_PS = max(_G, 8)
_PSH = _PS.bit_length() - 1
_GTAB = (_CHUNK // _G) * _PS
_STAB = ((_CHUNK + _L) // _G) * _PS


def _mixer_body(ids_hbm, emb_hbm, src_hbm, out_hbm, ids_v, gidx, sidx,
                *rest):
    bufs = rest[:_K]
    gsems = rest[_K:2 * _K]
    ssems = rest[2 * _K:]
    wid = lax.axis_index("s") * _NC + lax.axis_index("c")
    base_tok = wid * _CHUNK
    lanes = lax.iota(jnp.int32, _L)

    pltpu.sync_copy(ids_hbm, ids_v)

    def pre_body(j, acc):
        v = ids_v[pl.ds(j * _L, _L)]
        return acc + (v == _IMAGE_TOKEN_ID).astype(jnp.int32)

    acc = lax.fori_loop(0, wid * (_CHUNK // _L), pre_body,
                        jnp.zeros((_L,), jnp.int32))
    base_m = jnp.sum(acc)

    def scan1(j, carry):
        cnt, lmp, lup = carry
        v = ids_v[pl.ds(base_tok + j * _L, _L)]
        mi = (v == _IMAGE_TOKEN_ID).astype(jnp.int32)
        gpos = base_tok + j * _L + lanes
        lmp = jnp.maximum(lmp, jnp.max(jnp.where(mi == 1, gpos, -1)))
        lup = jnp.maximum(lup, jnp.max(jnp.where(mi == 1, -1, gpos)))
        return cnt + jnp.sum(mi), lmp, lup

    cnt_m, lmp, lup = lax.fori_loop(
        0, _CHUNK // _L, scan1,
        (jnp.int32(0), jnp.int32(-1), jnp.int32(-1)))
    cnt_u = _CHUNK - cnt_m
    tm = (cnt_m + _G - 1) >> _GS
    tu = (cnt_u + _G - 1) >> _GS

    def fill(j, _):
        pos = j * _L + lanes
        k = ((pos >> _PSH) << _GS) + (pos & (_G - 1))
        sidx[pl.ds(j * _L, _L)] = jnp.where(pos < tm * _PS, lmp, lup)
        @pl.when(j * _L < _GTAB)
        def _():
            gidx[pl.ds(j * _L, _L)] = base_m + jnp.minimum(k, cnt_m - 1)
        return 0

    lax.fori_loop(0, _STAB // _L, fill, 0)

    def scan2(j, c):
        v = ids_v[pl.ds(base_tok + j * _L, _L)]
        mi = (v == _IMAGE_TOKEN_ID).astype(jnp.int32)
        csum = plsc.cumsum(mi)
        gpos = base_tok + j * _L + lanes
        lr = jnp.clip(c + csum - 1, 0, jnp.maximum(cnt_m - 1, 0))
        ur = jnp.clip(j * _L + lanes - (c + csum), 0,
                      jnp.maximum(cnt_u - 1, 0))
        lrp = ((lr >> _GS) * _PS) + (lr & (_G - 1))
        urp = tm * _PS + ((ur >> _GS) * _PS) + (ur & (_G - 1))
        plsc.store_scatter(sidx, [lrp], gpos, mask=(mi == 1))
        plsc.store_scatter(sidx, [urp], gpos, mask=(mi == 0))
        return c + jnp.max(csum)

    lax.fori_loop(0, _CHUNK // _L, scan2, jnp.int32(0))

    trips = tm + tu

    def drain_s(b):
        pltpu.make_async_copy(bufs[b], out_hbm.at[pl.ds(0, _G)],
                              ssems[b]).wait()

    def drain_g(b):
        pltpu.make_async_copy(src_hbm.at[pl.ds(0, _G)], bufs[b],
                              gsems[b]).wait()

    def gather(g, b):
        @pl.when(g < tm)
        def _():
            pltpu.async_copy(src_hbm.at[gidx.at[pl.ds(g * _PS, _G)]],
                             bufs[b], gsems[b])

        @pl.when(g >= tm)
        def _():
            pltpu.async_copy(emb_hbm.at[sidx.at[pl.ds(g * _PS, _G)]],
                             bufs[b], gsems[b])

    def slot(g, b, bp):
        @pl.when(g < trips)
        def _():
            @pl.when(g >= _K)
            def _():
                drain_s(b)

            gather(g, b)

            @pl.when(g >= 1)
            def _():
                drain_g(bp)
                pltpu.async_copy(
                    bufs[bp], out_hbm.at[sidx.at[pl.ds((g - 1) * _PS, _G)]],
                    ssems[bp])

    def kslots(t, _):
        g = t * _K
        for i in range(_K):
            slot(g + i, i, (i - 1) % _K)
        return 0

    lax.fori_loop(0, (trips + _K - 1) // _K, kslots, 0)

    last = trips - 1
    for r in range(_K):
        @pl.when((trips >= 1) & (last % _K == r))
        def _():
            drain_g(r)
            pltpu.async_copy(
                bufs[r], out_hbm.at[sidx.at[pl.ds(last * _PS, _G)]],
                ssems[r])

    for r in range(_K):
        @pl.when(trips >= r + 1)
        def _():
            drain_s(r)


@functools.cache
def _mixer():
    return pl.kernel(
        _mixer_body,
        out_type=jax.ShapeDtypeStruct((_N, _D), jnp.float32),
        mesh=plsc.VectorSubcoreMesh(core_axis_name="c", subcore_axis_name="s",
                                    num_cores=_NC, num_subcores=_NS),
        scratch_types=[
            pltpu.VMEM((_N,), jnp.int32),
            pltpu.VMEM((_GTAB,), jnp.int32),
            pltpu.VMEM((_STAB,), jnp.int32),
        ] + [pltpu.VMEM((_G, _D), jnp.float32)] * _K
          + [pltpu.SemaphoreType.DMA] * (2 * _K),
        compiler_params=pltpu.CompilerParams(needs_layout_passes=False),
    )


def kernel(input_ids, inputs_embeds, image_features):
    B, S, D = inputs_embeds.shape
    ids = input_ids.reshape(B * S).astype(jnp.int32)
    emb = inputs_embeds.reshape(B * S, D)
    src = image_features.reshape(B * S, D)
    out = _mixer()(ids, emb, src)
    return out.reshape(B, S, D)


# speedup vs baseline: 1.0685x; 1.0685x over previous
import functools

import jax
import jax.numpy as jnp
from jax import lax
from jax.experimental import pallas as pl
from jax.experimental.pallas import tpu as pltpu
from jax.experimental.pallas import tpu_sc as plsc

_IMAGE_TOKEN_ID = 1

_N = 16384
_D = 2048
_NC = 2
_NS = 16
_NW = _NC * _NS
_CHUNK = _N // _NW
_L = 16
_GS = 3
_G = 1 << _GS
_NG = _CHUNK // _G
_K = 6
_PS = max(_G, 8)
_PSH = _PS.bit_length() - 1
_GTAB = (_CHUNK // _G) * _PS
_STAB = ((_CHUNK + _L) // _G) * _PS


def _mixer_body(ids_hbm, emb_hbm, src_hbm, out_hbm, ids_v, gidx, sidx,
                *rest):
    bufs = rest[:_K]
    gsems = rest[_K:2 * _K]
    ssems = rest[2 * _K:]
    wid = lax.axis_index("s") * _NC + lax.axis_index("c")
    base_tok = wid * _CHUNK
    lanes = lax.iota(jnp.int32, _L)

    pltpu.sync_copy(ids_hbm, ids_v)

    def pre_body(j, acc):
        v = ids_v[pl.ds(j * _L, _L)]
        return acc + (v == _IMAGE_TOKEN_ID).astype(jnp.int32)

    acc = lax.fori_loop(0, wid * (_CHUNK // _L), pre_body,
                        jnp.zeros((_L,), jnp.int32))
    base_m = jnp.sum(acc)

    def scan1(j, carry):
        cnt, lmp, lup = carry
        v = ids_v[pl.ds(base_tok + j * _L, _L)]
        mi = (v == _IMAGE_TOKEN_ID).astype(jnp.int32)
        gpos = base_tok + j * _L + lanes
        lmp = jnp.maximum(lmp, jnp.max(jnp.where(mi == 1, gpos, -1)))
        lup = jnp.maximum(lup, jnp.max(jnp.where(mi == 1, -1, gpos)))
        return cnt + jnp.sum(mi), lmp, lup

    cnt_m, lmp, lup = lax.fori_loop(
        0, _CHUNK // _L, scan1,
        (jnp.int32(0), jnp.int32(-1), jnp.int32(-1)))
    cnt_u = _CHUNK - cnt_m
    tm = (cnt_m + _G - 1) >> _GS
    tu = (cnt_u + _G - 1) >> _GS

    def fill(j, _):
        pos = j * _L + lanes
        k = ((pos >> _PSH) << _GS) + (pos & (_G - 1))
        sidx[pl.ds(j * _L, _L)] = jnp.where(pos < tm * _PS, lmp, lup)
        @pl.when(j * _L < _GTAB)
        def _():
            gidx[pl.ds(j * _L, _L)] = base_m + jnp.minimum(k, cnt_m - 1)
        return 0

    lax.fori_loop(0, _STAB // _L, fill, 0)

    def scan2(j, c):
        v = ids_v[pl.ds(base_tok + j * _L, _L)]
        mi = (v == _IMAGE_TOKEN_ID).astype(jnp.int32)
        csum = plsc.cumsum(mi)
        gpos = base_tok + j * _L + lanes
        lr = jnp.clip(c + csum - 1, 0, jnp.maximum(cnt_m - 1, 0))
        ur = jnp.clip(j * _L + lanes - (c + csum), 0,
                      jnp.maximum(cnt_u - 1, 0))
        lrp = ((lr >> _GS) * _PS) + (lr & (_G - 1))
        urp = tm * _PS + ((ur >> _GS) * _PS) + (ur & (_G - 1))
        plsc.store_scatter(sidx, [lrp], gpos, mask=(mi == 1))
        plsc.store_scatter(sidx, [urp], gpos, mask=(mi == 0))
        return c + jnp.max(csum)

    lax.fori_loop(0, _CHUNK // _L, scan2, jnp.int32(0))

    trips = tm + tu

    def drain_s(b):
        pltpu.make_async_copy(bufs[b], out_hbm.at[pl.ds(0, _G)],
                              ssems[b]).wait()

    def drain_g(b):
        pltpu.make_async_copy(src_hbm.at[pl.ds(0, _G)], bufs[b],
                              gsems[b]).wait()

    def gather(g, b):
        @pl.when(g < tm)
        def _():
            pltpu.async_copy(src_hbm.at[gidx.at[pl.ds(g * _PS, _G)]],
                             bufs[b], gsems[b])

        @pl.when(g >= tm)
        def _():
            pltpu.async_copy(emb_hbm.at[sidx.at[pl.ds(g * _PS, _G)]],
                             bufs[b], gsems[b])

    def scatter(g, b):
        drain_g(b)
        pltpu.async_copy(bufs[b], out_hbm.at[sidx.at[pl.ds(g * _PS, _G)]],
                         ssems[b])

    def slot(g, b, bpp):
        @pl.when(g < trips)
        def _():
            @pl.when(g >= _K)
            def _():
                drain_s(b)

            gather(g, b)

            @pl.when(g >= 2)
            def _():
                scatter(g - 2, bpp)

    def kslots(t, _):
        g = t * _K
        for i in range(_K):
            slot(g + i, i, (i - 2) % _K)
        return 0

    lax.fori_loop(0, (trips + _K - 1) // _K, kslots, 0)

    last = trips - 1
    for r in range(_K):
        @pl.when((trips >= 2) & ((last - 1) % _K == r))
        def _():
            scatter(last - 1, r)

    for r in range(_K):
        @pl.when((trips >= 1) & (last % _K == r))
        def _():
            scatter(last, r)

    for r in range(_K):
        @pl.when(trips >= r + 1)
        def _():
            drain_s(r)


@functools.cache
def _mixer():
    return pl.kernel(
        _mixer_body,
        out_type=jax.ShapeDtypeStruct((_N, _D), jnp.float32),
        mesh=plsc.VectorSubcoreMesh(core_axis_name="c", subcore_axis_name="s",
                                    num_cores=_NC, num_subcores=_NS),
        scratch_types=[
            pltpu.VMEM((_N,), jnp.int32),
            pltpu.VMEM((_GTAB,), jnp.int32),
            pltpu.VMEM((_STAB,), jnp.int32),
        ] + [pltpu.VMEM((_G, _D), jnp.float32)] * _K
          + [pltpu.SemaphoreType.DMA] * (2 * _K),
        compiler_params=pltpu.CompilerParams(needs_layout_passes=False),
    )


def kernel(input_ids, inputs_embeds, image_features):
    B, S, D = inputs_embeds.shape
    ids = input_ids.reshape(B * S).astype(jnp.int32)
    emb = inputs_embeds.reshape(B * S, D)
    src = image_features.reshape(B * S, D)
    out = _mixer()(ids, emb, src)
    return out.reshape(B, S, D)
